# baseline (speedup 1.0000x reference)
import jax
import jax.numpy as jnp
from jax.experimental import pallas as pl
from jax.experimental.pallas import tpu as pltpu

_B, _C, _H, _W = 4, 192, 32, 32
_T = _H * _W
_E = 8
_K = 2
_HID = _C * 4


def _top2(logits):
    iota = jax.lax.broadcasted_iota(jnp.int32, logits.shape, 1)
    m1 = jnp.max(logits, axis=1, keepdims=True)
    i1 = jnp.min(jnp.where(logits == m1, iota, _E), axis=1, keepdims=True)
    masked = jnp.where(iota == i1, -jnp.inf, logits)
    m2 = jnp.max(masked, axis=1, keepdims=True)
    i2 = jnp.min(jnp.where(masked == m2, iota, _E), axis=1, keepdims=True)
    e = jnp.exp(m2 - m1)
    g1 = 1.0 / (1.0 + e)
    g2 = e / (1.0 + e)
    return i1, i2, g1, g2


def _route_kernel(x_ref, y_ref, wlt_ref, wgt_ref, idx_ref, gate_ref):
    s = (jnp.sum(x_ref[...], axis=1)
         + jnp.sum(y_ref[...], axis=1)) * (0.5 / _T)
    ll = jax.lax.dot_general(s, wlt_ref[...], (((1,), (1,)), ((), ())),
                             preferred_element_type=jnp.float32)
    lg = jax.lax.dot_general(s, wgt_ref[...], (((1,), (1,)), ((), ())),
                             preferred_element_type=jnp.float32)
    li1, li2, lg1, lg2 = _top2(ll)
    gi1, gi2, gg1, gg2 = _top2(lg)
    idx_ref[...] = jnp.concatenate([li1, li2, gi1, gi2], axis=1)
    gate_ref[...] = jnp.concatenate([lg1, lg2, gg1, gg2], axis=1)


def _ffn(w1, b1row, w2t, b2row, inp):
    h = jax.lax.dot_general(inp.astype(jnp.bfloat16), w1.astype(jnp.bfloat16),
                            (((1,), (0,)), ((), ())),
                            preferred_element_type=jnp.float32)
    h = jnp.maximum(h + b1row, 0.0)
    o = jax.lax.dot_general(h.astype(jnp.bfloat16), w2t.astype(jnp.bfloat16),
                            (((1,), (1,)), ((), ())),
                            preferred_element_type=jnp.float32)
    return o + b2row


def _expert_kernel(idx_ref, gate_ref, x_ref, y_ref, lw1_ref, lb1_ref, lw2_ref,
                   lb2_ref, gw1_ref, gb1_ref, gw2_ref, gb2_ref, o_ref):
    b = pl.program_id(0)
    s = pl.program_id(1)
    il = idx_ref[b, s]
    ig = idx_ref[b, _K + s]
    xt = x_ref[0]
    fu = (xt + y_ref[0]) * 0.5
    ol = _ffn(lw1_ref[0], lb1_ref[pl.ds(il, 1), :], lw2_ref[0],
              lb2_ref[pl.ds(il, 1), :], xt)
    og = _ffn(gw1_ref[0], gb1_ref[pl.ds(ig, 1), :], gw2_ref[0],
              gb2_ref[pl.ds(ig, 1), :], fu)
    acc = gate_ref[b, s] * ol + gate_ref[b, _K + s] * og

    @pl.when(s == 0)
    def _init():
        o_ref[0] = acc

    @pl.when(s != 0)
    def _acc():
        o_ref[0] += acc


def kernel(x, y, w_gate_local, w_gate_global, lW1, lb1, lW2, lb2, gW1, gb1,
           gW2, gb2):
    xr = jnp.transpose(x, (0, 2, 3, 1)).reshape(_B, _T, _C)
    yr = jnp.transpose(y, (0, 2, 3, 1)).reshape(_B, _T, _C)
    lW2t = jnp.swapaxes(lW2, 1, 2)
    gW2t = jnp.swapaxes(gW2, 1, 2)

    idx, gates = pl.pallas_call(
        _route_kernel,
        out_shape=(
            jax.ShapeDtypeStruct((_B, 2 * _K), jnp.int32),
            jax.ShapeDtypeStruct((_B, 2 * _K), jnp.float32),
        ),
    )(xr, yr, w_gate_local.T, w_gate_global.T)

    full = lambda b, s, idx, g: (0, 0)
    grid = (_B, _K)
    out = pl.pallas_call(
        _expert_kernel,
        grid_spec=pltpu.PrefetchScalarGridSpec(
            num_scalar_prefetch=2,
            grid=grid,
            in_specs=[
                pl.BlockSpec((1, _T, _C), lambda b, s, idx, g: (b, 0, 0)),
                pl.BlockSpec((1, _T, _C), lambda b, s, idx, g: (b, 0, 0)),
                pl.BlockSpec((1, _C, _HID),
                             lambda b, s, idx, g: (idx[b, s], 0, 0)),
                pl.BlockSpec((_E, _HID), full),
                pl.BlockSpec((1, _C, _HID),
                             lambda b, s, idx, g: (idx[b, s], 0, 0)),
                pl.BlockSpec((_E, _C), full),
                pl.BlockSpec((1, _C, _HID),
                             lambda b, s, idx, g: (idx[b, _K + s], 0, 0)),
                pl.BlockSpec((_E, _HID), full),
                pl.BlockSpec((1, _C, _HID),
                             lambda b, s, idx, g: (idx[b, _K + s], 0, 0)),
                pl.BlockSpec((_E, _C), full),
            ],
            out_specs=pl.BlockSpec((1, _T, _C), lambda b, s, idx, g: (b, 0, 0)),
        ),
        out_shape=jax.ShapeDtypeStruct((_B, _T, _C), jnp.float32),
    )(idx, gates, xr, yr, lW1, lb1, lW2t, lb2, gW1, gb1, gW2t, gb2)

    return jnp.transpose(out.reshape(_B, _H, _W, _C), (0, 3, 1, 2))


# speedup vs baseline: 1.0037x; 1.0037x over previous
import jax
import jax.numpy as jnp
from jax.experimental import pallas as pl
from jax.experimental.pallas import tpu as pltpu

_B, _C, _H, _W = 4, 192, 32, 32
_T = _H * _W
_E = 8
_K = 2
_HID = _C * 4


def _top2(logits):
    iota = jax.lax.broadcasted_iota(jnp.int32, logits.shape, 1)
    m1 = jnp.max(logits, axis=1, keepdims=True)
    i1 = jnp.min(jnp.where(logits == m1, iota, _E), axis=1, keepdims=True)
    masked = jnp.where(iota == i1, -jnp.inf, logits)
    m2 = jnp.max(masked, axis=1, keepdims=True)
    i2 = jnp.min(jnp.where(masked == m2, iota, _E), axis=1, keepdims=True)
    e = jnp.exp(m2 - m1)
    g1 = 1.0 / (1.0 + e)
    g2 = e / (1.0 + e)
    return i1, i2, g1, g2


def _route_kernel(x_ref, y_ref, wlt_ref, wgt_ref, idx_ref, gate_ref):
    s = (jnp.sum(x_ref[...], axis=1)
         + jnp.sum(y_ref[...], axis=1)) * (0.5 / _T)
    ll = jax.lax.dot_general(s, wlt_ref[...], (((1,), (1,)), ((), ())),
                             preferred_element_type=jnp.float32)
    lg = jax.lax.dot_general(s, wgt_ref[...], (((1,), (1,)), ((), ())),
                             preferred_element_type=jnp.float32)
    li1, li2, lg1, lg2 = _top2(ll)
    gi1, gi2, gg1, gg2 = _top2(lg)
    idx_ref[...] = jnp.concatenate([li1, li2, gi1, gi2], axis=1)
    gate_ref[...] = jnp.concatenate([lg1, lg2, gg1, gg2], axis=1)


def _ffn(w1, b1row, w2t, b2row, inp):
    h = jax.lax.dot_general(inp, w1, (((1,), (0,)), ((), ())),
                            precision=jax.lax.Precision.DEFAULT,
                            preferred_element_type=jnp.float32)
    h = jnp.maximum(h + b1row, 0.0)
    o = jax.lax.dot_general(h, w2t, (((1,), (1,)), ((), ())),
                            precision=jax.lax.Precision.DEFAULT,
                            preferred_element_type=jnp.float32)
    return o + b2row


def _expert_kernel(idx_ref, gate_ref, x_ref, y_ref, lw1_ref, lb1_ref, lw2_ref,
                   lb2_ref, gw1_ref, gb1_ref, gw2_ref, gb2_ref, o_ref):
    b = pl.program_id(0)
    s = pl.program_id(1)
    il = idx_ref[b, s]
    ig = idx_ref[b, _K + s]
    xt = x_ref[0]
    fu = (xt + y_ref[0]) * 0.5
    ol = _ffn(lw1_ref[0], lb1_ref[pl.ds(il, 1), :], lw2_ref[0],
              lb2_ref[pl.ds(il, 1), :], xt)
    og = _ffn(gw1_ref[0], gb1_ref[pl.ds(ig, 1), :], gw2_ref[0],
              gb2_ref[pl.ds(ig, 1), :], fu)
    acc = gate_ref[b, s] * ol + gate_ref[b, _K + s] * og

    @pl.when(s == 0)
    def _init():
        o_ref[0] = acc

    @pl.when(s != 0)
    def _acc():
        o_ref[0] += acc


def kernel(x, y, w_gate_local, w_gate_global, lW1, lb1, lW2, lb2, gW1, gb1,
           gW2, gb2):
    xr = jnp.transpose(x, (0, 2, 3, 1)).reshape(_B, _T, _C)
    yr = jnp.transpose(y, (0, 2, 3, 1)).reshape(_B, _T, _C)
    lW2t = jnp.swapaxes(lW2, 1, 2)
    gW2t = jnp.swapaxes(gW2, 1, 2)

    idx, gates = pl.pallas_call(
        _route_kernel,
        out_shape=(
            jax.ShapeDtypeStruct((_B, 2 * _K), jnp.int32),
            jax.ShapeDtypeStruct((_B, 2 * _K), jnp.float32),
        ),
    )(xr, yr, w_gate_local.T, w_gate_global.T)

    full = lambda b, s, idx, g: (0, 0)
    grid = (_B, _K)
    out = pl.pallas_call(
        _expert_kernel,
        grid_spec=pltpu.PrefetchScalarGridSpec(
            num_scalar_prefetch=2,
            grid=grid,
            in_specs=[
                pl.BlockSpec((1, _T, _C), lambda b, s, idx, g: (b, 0, 0)),
                pl.BlockSpec((1, _T, _C), lambda b, s, idx, g: (b, 0, 0)),
                pl.BlockSpec((1, _C, _HID),
                             lambda b, s, idx, g: (idx[b, s], 0, 0)),
                pl.BlockSpec((_E, _HID), full),
                pl.BlockSpec((1, _C, _HID),
                             lambda b, s, idx, g: (idx[b, s], 0, 0)),
                pl.BlockSpec((_E, _C), full),
                pl.BlockSpec((1, _C, _HID),
                             lambda b, s, idx, g: (idx[b, _K + s], 0, 0)),
                pl.BlockSpec((_E, _HID), full),
                pl.BlockSpec((1, _C, _HID),
                             lambda b, s, idx, g: (idx[b, _K + s], 0, 0)),
                pl.BlockSpec((_E, _C), full),
            ],
            out_specs=pl.BlockSpec((1, _T, _C), lambda b, s, idx, g: (b, 0, 0)),
        ),
        out_shape=jax.ShapeDtypeStruct((_B, _T, _C), jnp.float32),
    )(idx, gates, xr, yr, lW1, lb1, lW2t, lb2, gW1, gb1, gW2t, gb2)

    return jnp.transpose(out.reshape(_B, _H, _W, _C), (0, 3, 1, 2))


# speedup vs baseline: 1.0466x; 1.0428x over previous
import jax
import jax.numpy as jnp
from jax.experimental import pallas as pl
from jax.experimental.pallas import tpu as pltpu

_B, _C, _H, _W = 4, 192, 32, 32
_T = _H * _W
_E = 8
_K = 2
_HID = _C * 4


def _top2(logits):
    iota = jax.lax.broadcasted_iota(jnp.int32, logits.shape, 1)
    m1 = jnp.max(logits, axis=1, keepdims=True)
    i1 = jnp.min(jnp.where(logits == m1, iota, _E), axis=1, keepdims=True)
    masked = jnp.where(iota == i1, -jnp.inf, logits)
    m2 = jnp.max(masked, axis=1, keepdims=True)
    i2 = jnp.min(jnp.where(masked == m2, iota, _E), axis=1, keepdims=True)
    e = jnp.exp(m2 - m1)
    g1 = 1.0 / (1.0 + e)
    g2 = e / (1.0 + e)
    return i1, i2, g1, g2


def _route_kernel(x_ref, y_ref, wlt_ref, wgt_ref, idx_ref, gate_ref):
    s = (jnp.sum(x_ref[...], axis=1)
         + jnp.sum(y_ref[...], axis=1)) * (0.5 / _T)
    ll = jax.lax.dot_general(s, wlt_ref[...], (((1,), (1,)), ((), ())),
                             preferred_element_type=jnp.float32)
    lg = jax.lax.dot_general(s, wgt_ref[...], (((1,), (1,)), ((), ())),
                             preferred_element_type=jnp.float32)
    li1, li2, lg1, lg2 = _top2(ll)
    gi1, gi2, gg1, gg2 = _top2(lg)
    idx_ref[...] = jnp.concatenate([li1, li2, gi1, gi2], axis=1)
    gate_ref[...] = jnp.concatenate([lg1, lg2, gg1, gg2], axis=1)


def _ffn(w1, b1row, w2t, b2row, inp):
    h = jax.lax.dot_general(inp, w1, (((1,), (0,)), ((), ())),
                            precision=jax.lax.Precision.DEFAULT,
                            preferred_element_type=jnp.float32)
    h = jnp.maximum(h + b1row, 0.0)
    o = jax.lax.dot_general(h, w2t, (((1,), (1,)), ((), ())),
                            precision=jax.lax.Precision.DEFAULT,
                            preferred_element_type=jnp.float32)
    return o + b2row


def _expert_kernel(idx_ref, gate_ref, x_ref, y_ref, lw1a_ref, lw1b_ref,
                   lw2a_ref, lw2b_ref, lb1_ref, lb2_ref, gw1a_ref, gw1b_ref,
                   gw2a_ref, gw2b_ref, gb1_ref, gb2_ref, o_ref):
    b = pl.program_id(0)
    il0 = idx_ref[b, 0]
    il1 = idx_ref[b, 1]
    ig0 = idx_ref[b, 2]
    ig1 = idx_ref[b, 3]
    xt = x_ref[0]
    fu = (xt + y_ref[0]) * 0.5
    ol0 = _ffn(lw1a_ref[0], lb1_ref[pl.ds(il0, 1), :], lw2a_ref[0],
               lb2_ref[pl.ds(il0, 1), :], xt)
    og0 = _ffn(gw1a_ref[0], gb1_ref[pl.ds(ig0, 1), :], gw2a_ref[0],
               gb2_ref[pl.ds(ig0, 1), :], fu)
    ol1 = _ffn(lw1b_ref[0], lb1_ref[pl.ds(il1, 1), :], lw2b_ref[0],
               lb2_ref[pl.ds(il1, 1), :], xt)
    og1 = _ffn(gw1b_ref[0], gb1_ref[pl.ds(ig1, 1), :], gw2b_ref[0],
               gb2_ref[pl.ds(ig1, 1), :], fu)
    o_ref[0] = (gate_ref[b, 0] * ol0 + gate_ref[b, 1] * ol1
                + gate_ref[b, 2] * og0 + gate_ref[b, 3] * og1)


def kernel(x, y, w_gate_local, w_gate_global, lW1, lb1, lW2, lb2, gW1, gb1,
           gW2, gb2):
    xr = jnp.transpose(x, (0, 2, 3, 1)).reshape(_B, _T, _C)
    yr = jnp.transpose(y, (0, 2, 3, 1)).reshape(_B, _T, _C)
    lW2t = jnp.swapaxes(lW2, 1, 2)
    gW2t = jnp.swapaxes(gW2, 1, 2)

    idx, gates = pl.pallas_call(
        _route_kernel,
        out_shape=(
            jax.ShapeDtypeStruct((_B, 2 * _K), jnp.int32),
            jax.ShapeDtypeStruct((_B, 2 * _K), jnp.float32),
        ),
    )(xr, yr, w_gate_local.T, w_gate_global.T)

    full = lambda b, idx, g: (0, 0)
    wsel = lambda col: (lambda b, idx, g: (idx[b, col], 0, 0))
    grid = (_B,)
    out = pl.pallas_call(
        _expert_kernel,
        grid_spec=pltpu.PrefetchScalarGridSpec(
            num_scalar_prefetch=2,
            grid=grid,
            in_specs=[
                pl.BlockSpec((1, _T, _C), lambda b, idx, g: (b, 0, 0)),
                pl.BlockSpec((1, _T, _C), lambda b, idx, g: (b, 0, 0)),
                pl.BlockSpec((1, _C, _HID), wsel(0)),
                pl.BlockSpec((1, _C, _HID), wsel(1)),
                pl.BlockSpec((1, _C, _HID), wsel(0)),
                pl.BlockSpec((1, _C, _HID), wsel(1)),
                pl.BlockSpec((_E, _HID), full),
                pl.BlockSpec((_E, _C), full),
                pl.BlockSpec((1, _C, _HID), wsel(2)),
                pl.BlockSpec((1, _C, _HID), wsel(3)),
                pl.BlockSpec((1, _C, _HID), wsel(2)),
                pl.BlockSpec((1, _C, _HID), wsel(3)),
                pl.BlockSpec((_E, _HID), full),
                pl.BlockSpec((_E, _C), full),
            ],
            out_specs=pl.BlockSpec((1, _T, _C), lambda b, idx, g: (b, 0, 0)),
        ),
        out_shape=jax.ShapeDtypeStruct((_B, _T, _C), jnp.float32),
    )(idx, gates, xr, yr, lW1, lW1, lW2t, lW2t, lb1, lb2,
      gW1, gW1, gW2t, gW2t, gb1, gb2)

    return jnp.transpose(out.reshape(_B, _H, _W, _C), (0, 3, 1, 2))
